# baseline (speedup 1.0000x reference)
import jax
import jax.numpy as jnp
from jax import lax
from jax.experimental import pallas as pl
from jax.experimental.pallas import tpu as pltpu
from jax.experimental.pallas import tpu_sc as plsc

N_NODES = 10000
N_EDGES = 320000
FEAT = 128
NUM_RELS = 200
NUM_BASES = 8
SUB = 16

N_PAD = 10240
ROWS = 320
SCAN_CHUNK = 3200
N_SCAN = N_EDGES // SCAN_CHUNK
M_CAP = 12288
M_WIN = M_CAP // 128
P_CAP = 16384
SENT = N_EDGES
TYPE_PAD = 255

_BCAST_DNUMS = lax.GatherDimensionNumbers(
    offset_dims=(), collapsed_slice_dims=(0,), start_index_map=(0,))


def _bcast(vec, idx):
    iv = jnp.full((SUB, 1), idx, jnp.int32)
    return lax.gather(vec, iv, _BCAST_DNUMS, (1,),
                      mode=lax.GatherScatterMode.PROMISE_IN_BOUNDS)


def _body(x_hbm, edata_hbm, dst_hbm, norm_hbm, w_hbm, out_hbm,
          h, mids, mdst, gspk, gsrcg, gdstg, dstbuf, wbuf, xg, nbuf,
          hist, cur, ctype, semg, sem0, sem1):
    c = lax.axis_index("c")
    s = lax.axis_index("s")
    wid = c * 16 + s
    lo = wid * ROWS
    hi = lo + ROWS
    zero16f = jnp.zeros((SUB,), jnp.float32)
    sent16 = jnp.full((SUB,), SENT, jnp.int32)
    iota16 = lax.iota(jnp.int32, SUB)
    zeros16 = jnp.zeros((SUB,), jnp.int32)
    ones16 = jnp.full((SUB,), 1, jnp.int32)

    _sc = jax.named_scope("p_init"); _sc.__enter__()
    def _zero_h(r, _):
        for j in range(FEAT // SUB):
            h[r, pl.ds(j * SUB, SUB)] = zero16f
        return 0
    lax.fori_loop(0, ROWS, _zero_h, 0)

    def _fill_mids(g, _):
        mids[pl.ds(g * SUB, SUB)] = sent16
        return 0
    lax.fori_loop(0, M_CAP // SUB, _fill_mids, 0)

    def _fill_gd(g, _):
        gdstg[pl.ds(g * SUB, SUB)] = jnp.full((SUB,), N_PAD, jnp.int32)
        gsrcg[pl.ds(g * SUB, SUB)] = zeros16
        return 0
    lax.fori_loop(0, P_CAP // SUB, _fill_gd, 0)

    def _hz(t, _):
        hist[t] = 0
        return 0
    lax.fori_loop(0, 256, _hz, 0)
    _sc.__exit__(None, None, None)

    _sc = jax.named_scope("p_scan"); _sc.__enter__()
    def _scan_chunk(k, m):
        pltpu.sync_copy(dst_hbm.at[pl.ds(k * SCAN_CHUNK, SCAN_CHUNK)], dstbuf)
        def vec_body(v, m):
            dv = dstbuf[pl.ds(v * SUB, SUB)]
            inr = (dv >= lo) & (dv < hi)
            eids = k * SCAN_CHUNK + v * SUB + iota16
            plsc.store_compressed(mids.at[pl.ds(m, SUB)], eids, mask=inr)
            plsc.store_compressed(mdst.at[pl.ds(m, SUB)], dv, mask=inr)
            cnt = plsc.all_reduce_population_count(inr)
            return jnp.minimum(m + cnt[0], M_CAP - SUB)
        return lax.fori_loop(0, SCAN_CHUNK // SUB, vec_body, m)
    m = lax.fori_loop(0, N_SCAN, _scan_chunk, jnp.int32(0))
    _sc.__exit__(None, None, None)

    _sc = jax.named_scope("p_gedata"); _sc.__enter__()
    cps = [pltpu.async_copy(
               edata_hbm.at[mids.at[pl.ds(g * 128, 128)]],
               gspk.at[pl.ds(g * 128, 128)], semg)
           for g in range(M_WIN)]
    for cp in cps:
        cp.wait()
    _sc.__exit__(None, None, None)

    ngrp = (m + SUB - 1) // SUB

    _sc = jax.named_scope("p_histgroup"); _sc.__enter__()
    def _hist_grp(g, _):
        sv = gspk[pl.ds(g * SUB, SUB)]
        tv = lax.shift_right_logical(sv, 16)
        for l in range(SUB):
            t = tv[l]
            hist[t] = hist[t] + 1
        return 0
    lax.fori_loop(0, ngrp, _hist_grp, 0)

    def _pfx(t, run):
        cnt = hist[t]
        hist[t] = run
        cur[t] = run
        return run + ((cnt + SUB - 1) // SUB) * SUB
    total = lax.fori_loop(0, NUM_RELS, _pfx, jnp.int32(0))
    hist[NUM_RELS] = total
    cur[TYPE_PAD] = total
    npc = total // SUB

    def _ct(t, _):
        c0 = hist[t] // SUB
        c1 = hist[t + 1] // SUB
        def w(cc, _):
            ctype[cc] = t
            return 0
        lax.fori_loop(c0, c1, w, 0)
        return 0
    lax.fori_loop(0, NUM_RELS, _ct, 0)

    lane0 = iota16 == 0
    def _group_grp(g, _):
        sv = gspk[pl.ds(g * SUB, SUB)]
        dvv = mdst[pl.ds(g * SUB, SUB)]
        tv = lax.shift_right_logical(sv, 16)
        srcv = jnp.bitwise_and(sv, 0xFFFF)
        for l in range(SUB):
            t = tv[l]
            p = cur[t]
            cur[t] = p + 1
            pf = jnp.full((SUB,), p, jnp.int32)
            plsc.store_scatter(gsrcg, [pf],
                               jnp.full((SUB,), srcv[l], jnp.int32), mask=lane0)
            plsc.store_scatter(gdstg, [pf],
                               jnp.full((SUB,), dvv[l], jnp.int32), mask=lane0)
        return 0
    lax.fori_loop(0, ngrp, _group_grp, 0)
    _sc.__exit__(None, None, None)

    _sc = jax.named_scope("p_compute"); _sc.__enter__()
    def _issue(cc, slot):
        t = ctype[cc]
        sem = (sem0, sem1)[slot]
        pltpu.async_copy(w_hbm.at[t], wbuf.at[slot], sem)
        pltpu.async_copy(
            x_hbm.at[gsrcg.at[pl.ds(cc * SUB, SUB)]], xg.at[slot], sem)

    def _wait(cc, slot):
        t = ctype[cc]
        sem = (sem0, sem1)[slot]
        pltpu.make_async_copy(w_hbm.at[t], wbuf.at[slot], sem).wait()
        pltpu.make_async_copy(
            x_hbm.at[gsrcg.at[pl.ds(cc * SUB, SUB)]], xg.at[slot], sem).wait()

    @pl.when(npc > 0)
    def _():
        _issue(jnp.int32(0), 0)

    def _chunk_body(cc, _):
        even = cc % 2 == 0
        slot = cc % 2
        @pl.when((cc + 1 < npc) & even)
        def _():
            _issue(cc + 1, 1)
        @pl.when((cc + 1 < npc) & jnp.logical_not(even))
        def _():
            _issue(cc + 1, 0)
        @pl.when(even)
        def _():
            _wait(cc, 0)
        @pl.when(jnp.logical_not(even))
        def _():
            _wait(cc, 1)

        dv = gdstg[pl.ds(cc * SUB, SUB)]
        dr_all = dv - lo
        okv = (dr_all >= 0) & (dr_all < ROWS)
        drc_all = jnp.where(okv, dr_all, ROWS)
        slotf = jnp.full((SUB,), slot, jnp.int32)
        for b in range(NUM_BASES):
            wr = [wbuf[slot, pl.ds(b * 256 + i * SUB, SUB)]
                  for i in range(SUB)]
            def _edge_body(e16, _):
                drowc = _bcast(drc_all, e16)
                e16f = jnp.full((SUB,), e16, jnp.int32)
                xb = xg[slot, e16, pl.ds(b * SUB, SUB)]
                def xbc(i):
                    if i % 2 == 0:
                        return plsc.load_gather(
                            xg, [slotf, e16f,
                                 jnp.full((SUB,), b * SUB + i, jnp.int32)])
                    return _bcast(xb, i)
                accs = [xbc(k) * wr[k] for k in range(8)]
                for i in range(8, SUB):
                    k = i % 8
                    accs[k] = accs[k] + xbc(i) * wr[i]
                acc = (((accs[0] + accs[1]) + (accs[2] + accs[3]))
                       + ((accs[4] + accs[5]) + (accs[6] + accs[7])))
                plsc.addupdate_scatter(h, [drowc, b * SUB + iota16], acc)
                return 0
            lax.fori_loop(0, SUB, _edge_body, 0)
        return 0
    lax.fori_loop(0, npc, _chunk_body, 0)
    _sc.__exit__(None, None, None)

    _sc = jax.named_scope("p_epi"); _sc.__enter__()
    pltpu.sync_copy(norm_hbm.at[pl.ds(lo, ROWS)], nbuf)
    def _norm_grp(g, _):
        nv = nbuf[pl.ds(g * SUB, SUB)]
        for l in range(SUB):
            nvs = nv[l]
            r = g * SUB + l
            for j in range(FEAT // SUB):
                h[r, pl.ds(j * SUB, SUB)] = h[r, pl.ds(j * SUB, SUB)] * nvs
        return 0
    lax.fori_loop(0, ROWS // SUB, _norm_grp, 0)
    pltpu.sync_copy(h.at[pl.ds(0, ROWS)], out_hbm.at[pl.ds(lo, ROWS)])
    _sc.__exit__(None, None, None)


@jax.jit
def _rgcn_sc(x, edata, dst, norm_pad, weight):
    mesh = plsc.VectorSubcoreMesh(core_axis_name="c", subcore_axis_name="s")
    f = pl.kernel(
        _body,
        out_type=jax.ShapeDtypeStruct((N_PAD, FEAT), jnp.float32),
        mesh=mesh,
        compiler_params=pltpu.CompilerParams(needs_layout_passes=False),
        scratch_types=[
            pltpu.VMEM((ROWS + 1, FEAT), jnp.float32),
            pltpu.VMEM((M_CAP,), jnp.int32),
            pltpu.VMEM((M_CAP,), jnp.int32),
            pltpu.VMEM((M_CAP,), jnp.int32),
            pltpu.VMEM((P_CAP,), jnp.int32),
            pltpu.VMEM((P_CAP,), jnp.int32),
            pltpu.VMEM((SCAN_CHUNK,), jnp.int32),
            pltpu.VMEM((2, NUM_BASES * SUB * SUB), jnp.float32),
            pltpu.VMEM((2, SUB, FEAT), jnp.float32),
            pltpu.VMEM((ROWS,), jnp.float32),
            pltpu.SMEM((256,), jnp.int32),
            pltpu.SMEM((256,), jnp.int32),
            pltpu.SMEM((1024,), jnp.int32),
            pltpu.SemaphoreType.DMA,
            pltpu.SemaphoreType.DMA,
            pltpu.SemaphoreType.DMA,
        ],
    )
    return f(x, edata, dst, norm_pad, weight)


def kernel(x, edge_index, edge_type, norm, weight):
    src = edge_index[0].astype(jnp.int32)
    dst = edge_index[1].astype(jnp.int32)
    et = edge_type.astype(jnp.int32)
    packed = src | (et << 16)
    edata = jnp.pad(packed, (0, SUB), constant_values=TYPE_PAD << 16)
    dst_pad = jnp.pad(dst, (0, SUB), constant_values=N_PAD)
    norm_pad = jnp.pad(norm[:, 0].astype(jnp.float32), (0, N_PAD - N_NODES))
    out = _rgcn_sc(x.astype(jnp.float32), edata, dst_pad, norm_pad,
                   weight.astype(jnp.float32))
    return out[:N_NODES]


# speedup vs baseline: 1.0084x; 1.0084x over previous
import jax
import jax.numpy as jnp
from jax import lax
from jax.experimental import pallas as pl
from jax.experimental.pallas import tpu as pltpu
from jax.experimental.pallas import tpu_sc as plsc

N_NODES = 10000
N_EDGES = 320000
FEAT = 128
NUM_RELS = 200
NUM_BASES = 8
SUB = 16

N_PAD = 10240
ROWS = 320
SCAN_CHUNK = 3200
N_SCAN = N_EDGES // SCAN_CHUNK
M_CAP = 12288
M_WIN = M_CAP // 128
P_CAP = 16384
SENT = N_EDGES
TYPE_PAD = 255

_BCAST_DNUMS = lax.GatherDimensionNumbers(
    offset_dims=(), collapsed_slice_dims=(0,), start_index_map=(0,))


def _bcast(vec, idx):
    iv = jnp.full((SUB, 1), idx, jnp.int32)
    return lax.gather(vec, iv, _BCAST_DNUMS, (1,),
                      mode=lax.GatherScatterMode.PROMISE_IN_BOUNDS)


def _body(x_hbm, edata_hbm, dst_hbm, norm_hbm, w_hbm, out_hbm,
          h, mids, mdst, gspk, gsrcg, gdstg, dstbuf, wbuf, xg, nbuf,
          hist, cur, ctype, semg, sem0, sem1):
    c = lax.axis_index("c")
    s = lax.axis_index("s")
    wid = c * 16 + s
    lo = wid * ROWS
    hi = lo + ROWS
    zero16f = jnp.zeros((SUB,), jnp.float32)
    sent16 = jnp.full((SUB,), SENT, jnp.int32)
    iota16 = lax.iota(jnp.int32, SUB)
    zeros16 = jnp.zeros((SUB,), jnp.int32)
    ones16 = jnp.full((SUB,), 1, jnp.int32)

    _sc = jax.named_scope("p_init"); _sc.__enter__()
    def _zero_h(r, _):
        for j in range(FEAT // SUB):
            h[r, pl.ds(j * SUB, SUB)] = zero16f
        return 0
    lax.fori_loop(0, ROWS, _zero_h, 0)

    def _fill_mids(g, _):
        mids[pl.ds(g * SUB, SUB)] = sent16
        return 0
    lax.fori_loop(0, M_CAP // SUB, _fill_mids, 0)

    def _fill_gd(g, _):
        gdstg[pl.ds(g * SUB, SUB)] = jnp.full((SUB,), N_PAD, jnp.int32)
        gsrcg[pl.ds(g * SUB, SUB)] = zeros16
        return 0
    lax.fori_loop(0, P_CAP // SUB, _fill_gd, 0)

    def _hz(t, _):
        hist[t] = 0
        return 0
    lax.fori_loop(0, 256, _hz, 0)
    _sc.__exit__(None, None, None)

    _sc = jax.named_scope("p_scan"); _sc.__enter__()
    def _scan_chunk(k, m):
        pltpu.sync_copy(dst_hbm.at[pl.ds(k * SCAN_CHUNK, SCAN_CHUNK)], dstbuf)
        def vec_body(v, m):
            dv = dstbuf[pl.ds(v * SUB, SUB)]
            inr = (dv >= lo) & (dv < hi)
            eids = k * SCAN_CHUNK + v * SUB + iota16
            plsc.store_compressed(mids.at[pl.ds(m, SUB)], eids, mask=inr)
            plsc.store_compressed(mdst.at[pl.ds(m, SUB)], dv, mask=inr)
            cnt = plsc.all_reduce_population_count(inr)
            return jnp.minimum(m + cnt[0], M_CAP - SUB)
        return lax.fori_loop(0, SCAN_CHUNK // SUB, vec_body, m)
    m = lax.fori_loop(0, N_SCAN, _scan_chunk, jnp.int32(0))
    _sc.__exit__(None, None, None)

    _sc = jax.named_scope("p_gedata"); _sc.__enter__()
    cps = [pltpu.async_copy(
               edata_hbm.at[mids.at[pl.ds(g * 128, 128)]],
               gspk.at[pl.ds(g * 128, 128)], semg)
           for g in range(M_WIN)]
    for cp in cps:
        cp.wait()
    _sc.__exit__(None, None, None)

    ngrp = (m + SUB - 1) // SUB

    _sc = jax.named_scope("p_histgroup"); _sc.__enter__()
    def _hist_grp(g, _):
        sv = gspk[pl.ds(g * SUB, SUB)]
        tv = lax.shift_right_logical(sv, 16)
        for l in range(SUB):
            t = tv[l]
            hist[t] = hist[t] + 1
        return 0
    lax.fori_loop(0, ngrp, _hist_grp, 0)

    def _pfx(t, run):
        cnt = hist[t]
        hist[t] = run
        cur[t] = run
        return run + ((cnt + SUB - 1) // SUB) * SUB
    total = lax.fori_loop(0, NUM_RELS, _pfx, jnp.int32(0))
    hist[NUM_RELS] = total
    cur[TYPE_PAD] = total
    npc = total // SUB

    def _ct(t, _):
        c0 = hist[t] // SUB
        c1 = hist[t + 1] // SUB
        def w(cc, _):
            ctype[cc] = t
            return 0
        lax.fori_loop(c0, c1, w, 0)
        return 0
    lax.fori_loop(0, NUM_RELS, _ct, 0)

    lane0 = iota16 == 0
    def _group_grp(g, _):
        sv = gspk[pl.ds(g * SUB, SUB)]
        dvv = mdst[pl.ds(g * SUB, SUB)]
        tv = lax.shift_right_logical(sv, 16)
        srcv = jnp.bitwise_and(sv, 0xFFFF)
        for l in range(SUB):
            t = tv[l]
            p = cur[t]
            cur[t] = p + 1
            pf = jnp.full((SUB,), p, jnp.int32)
            plsc.store_scatter(gsrcg, [pf],
                               jnp.full((SUB,), srcv[l], jnp.int32), mask=lane0)
            plsc.store_scatter(gdstg, [pf],
                               jnp.full((SUB,), dvv[l], jnp.int32), mask=lane0)
        return 0
    lax.fori_loop(0, ngrp, _group_grp, 0)
    _sc.__exit__(None, None, None)

    _sc = jax.named_scope("p_compute"); _sc.__enter__()
    def _issue(cc, slot):
        t = ctype[cc]
        sem = (sem0, sem1)[slot]
        pltpu.async_copy(w_hbm.at[t], wbuf.at[slot], sem)
        pltpu.async_copy(
            x_hbm.at[gsrcg.at[pl.ds(cc * SUB, SUB)]], xg.at[slot], sem)

    def _wait(cc, slot):
        t = ctype[cc]
        sem = (sem0, sem1)[slot]
        pltpu.make_async_copy(w_hbm.at[t], wbuf.at[slot], sem).wait()
        pltpu.make_async_copy(
            x_hbm.at[gsrcg.at[pl.ds(cc * SUB, SUB)]], xg.at[slot], sem).wait()

    @pl.when(npc > 0)
    def _():
        _issue(jnp.int32(0), 0)

    def _chunk_body(cc, _):
        even = cc % 2 == 0
        slot = cc % 2
        @pl.when((cc + 1 < npc) & even)
        def _():
            _issue(cc + 1, 1)
        @pl.when((cc + 1 < npc) & jnp.logical_not(even))
        def _():
            _issue(cc + 1, 0)
        @pl.when(even)
        def _():
            _wait(cc, 0)
        @pl.when(jnp.logical_not(even))
        def _():
            _wait(cc, 1)

        dv = gdstg[pl.ds(cc * SUB, SUB)]
        dr_all = dv - lo
        okv = (dr_all >= 0) & (dr_all < ROWS)
        drc_all = jnp.where(okv, dr_all, ROWS)
        slotf = jnp.full((SUB,), slot, jnp.int32)
        for b in range(NUM_BASES):
            wr = [wbuf[slot, pl.ds(b * 256 + i * SUB, SUB)]
                  for i in range(SUB)]
            @plsc.parallel_loop(0, SUB, unroll=2)
            def _edge_body(e16):
                drowc = _bcast(drc_all, e16)
                e16f = jnp.full((SUB,), e16, jnp.int32)
                xb = xg[slot, e16, pl.ds(b * SUB, SUB)]
                def xbc(i):
                    if i % 2 == 0:
                        return plsc.load_gather(
                            xg, [slotf, e16f,
                                 jnp.full((SUB,), b * SUB + i, jnp.int32)])
                    return _bcast(xb, i)
                accs = [xbc(k) * wr[k] for k in range(8)]
                for i in range(8, SUB):
                    k = i % 8
                    accs[k] = accs[k] + xbc(i) * wr[i]
                acc = (((accs[0] + accs[1]) + (accs[2] + accs[3]))
                       + ((accs[4] + accs[5]) + (accs[6] + accs[7])))
                plsc.addupdate_scatter(h, [drowc, b * SUB + iota16], acc)

        return 0
    lax.fori_loop(0, npc, _chunk_body, 0)
    _sc.__exit__(None, None, None)

    _sc = jax.named_scope("p_epi"); _sc.__enter__()
    pltpu.sync_copy(norm_hbm.at[pl.ds(lo, ROWS)], nbuf)
    def _norm_grp(g, _):
        nv = nbuf[pl.ds(g * SUB, SUB)]
        for l in range(SUB):
            nvs = nv[l]
            r = g * SUB + l
            for j in range(FEAT // SUB):
                h[r, pl.ds(j * SUB, SUB)] = h[r, pl.ds(j * SUB, SUB)] * nvs
        return 0
    lax.fori_loop(0, ROWS // SUB, _norm_grp, 0)
    pltpu.sync_copy(h.at[pl.ds(0, ROWS)], out_hbm.at[pl.ds(lo, ROWS)])
    _sc.__exit__(None, None, None)


@jax.jit
def _rgcn_sc(x, edata, dst, norm_pad, weight):
    mesh = plsc.VectorSubcoreMesh(core_axis_name="c", subcore_axis_name="s")
    f = pl.kernel(
        _body,
        out_type=jax.ShapeDtypeStruct((N_PAD, FEAT), jnp.float32),
        mesh=mesh,
        compiler_params=pltpu.CompilerParams(needs_layout_passes=False),
        scratch_types=[
            pltpu.VMEM((ROWS + 1, FEAT), jnp.float32),
            pltpu.VMEM((M_CAP,), jnp.int32),
            pltpu.VMEM((M_CAP,), jnp.int32),
            pltpu.VMEM((M_CAP,), jnp.int32),
            pltpu.VMEM((P_CAP,), jnp.int32),
            pltpu.VMEM((P_CAP,), jnp.int32),
            pltpu.VMEM((SCAN_CHUNK,), jnp.int32),
            pltpu.VMEM((2, NUM_BASES * SUB * SUB), jnp.float32),
            pltpu.VMEM((2, SUB, FEAT), jnp.float32),
            pltpu.VMEM((ROWS,), jnp.float32),
            pltpu.SMEM((256,), jnp.int32),
            pltpu.SMEM((256,), jnp.int32),
            pltpu.SMEM((1024,), jnp.int32),
            pltpu.SemaphoreType.DMA,
            pltpu.SemaphoreType.DMA,
            pltpu.SemaphoreType.DMA,
        ],
    )
    return f(x, edata, dst, norm_pad, weight)


def kernel(x, edge_index, edge_type, norm, weight):
    src = edge_index[0].astype(jnp.int32)
    dst = edge_index[1].astype(jnp.int32)
    et = edge_type.astype(jnp.int32)
    packed = src | (et << 16)
    edata = jnp.pad(packed, (0, SUB), constant_values=TYPE_PAD << 16)
    dst_pad = jnp.pad(dst, (0, SUB), constant_values=N_PAD)
    norm_pad = jnp.pad(norm[:, 0].astype(jnp.float32), (0, N_PAD - N_NODES))
    out = _rgcn_sc(x.astype(jnp.float32), edata, dst_pad, norm_pad,
                   weight.astype(jnp.float32))
    return out[:N_NODES]
